# baseline (speedup 1.0000x reference)
import functools

import jax
import jax.numpy as jnp
from jax import lax
from jax.experimental import pallas as pl
from jax.experimental.pallas import tpu as pltpu
from jax.experimental.pallas import tpu_sc as plsc

N_NODES = 10000
N_EDGES = 320000
D_FEAT = 128
D_OUT = 128

NC = 2
NS = 16
NW = NC * NS
ROW = 80
K = 125
CPAD = 10240
FIRE = 25

_mesh = plsc.VectorSubcoreMesh(core_axis_name="c", subcore_axis_name="s",
                               num_cores=NC)


EPW = K * ROW


@functools.partial(
    pl.kernel,
    out_type=jax.ShapeDtypeStruct((NC, CPAD), jnp.float32),
    mesh=_mesh,
    scratch_types=[
        pltpu.VMEM((EPW,), jnp.int32),
        pltpu.VMEM((112,), jnp.float32),
        pltpu.VMEM((CPAD // NS,), jnp.float32),
        pltpu.VMEM_SHARED((CPAD,), jnp.float32),
        pltpu.SemaphoreType.DMA,
    ],
)
def _hist_kernel(idx_hbm, out_hbm, idx_v, ones_v, zero_v, counts_sh, sem):
    cid = lax.axis_index("c")
    sid = lax.axis_index("s")
    wid = sid * NC + cid

    d_idx = pltpu.async_copy(idx_hbm.at[pl.ds(wid * EPW, EPW)], idx_v, sem)
    for i in range(112 // 16):
        ones_v[pl.ds(i * 16, 16)] = jnp.ones((16,), jnp.float32)
    stripe = CPAD // NS
    for i in range(stripe // 16):
        zero_v[pl.ds(i * 16, 16)] = jnp.zeros((16,), jnp.float32)
    pltpu.sync_copy(zero_v, counts_sh.at[pl.ds(sid * stripe, stripe)])

    plsc.subcore_barrier()
    d_idx.wait()

    ones_row = ones_v.at[pl.ds(0, ROW)]

    def _chunk(i, carry):
        descs = []
        for b in range(FIRE):
            r = i * FIRE + b
            descs.append(pltpu.async_copy(
                ones_row, counts_sh.at[idx_v.at[pl.ds(r * ROW, ROW)]],
                sem, add=True))
        for d in descs:
            d.wait()
        return carry

    lax.fori_loop(0, K // FIRE, _chunk, 0)

    plsc.subcore_barrier()

    pltpu.sync_copy(counts_sh.at[pl.ds(sid * stripe, stripe)],
                    out_hbm.at[cid, pl.ds(sid * stripe, stripe)])


BLK = 2048


def _scale_matmul_body(c_ref, x_ref, w_ref, b_ref, o_ref):
    y = lax.dot_general(
        x_ref[...], w_ref[...],
        (((1,), (1,)), ((), ())),
        preferred_element_type=jnp.float32,
    ) + b_ref[...]
    c_col = lax.dot_general(
        c_ref[...], jnp.ones((NC, 1), jnp.float32),
        (((0,), (0,)), ((), ())),
        preferred_element_type=jnp.float32,
    )
    o_ref[...] = y * c_col


def _scale_matmul(counts, x, w, b):
    return pl.pallas_call(
        _scale_matmul_body,
        grid=(pl.cdiv(N_NODES, BLK),),
        in_specs=[
            pl.BlockSpec((NC, BLK), lambda i: (0, i)),
            pl.BlockSpec((BLK, D_FEAT), lambda i: (i, 0)),
            pl.BlockSpec((D_OUT, D_FEAT), lambda i: (0, 0)),
            pl.BlockSpec((1, D_OUT), lambda i: (0, 0)),
        ],
        out_specs=pl.BlockSpec((BLK, D_OUT), lambda i: (i, 0)),
        out_shape=jax.ShapeDtypeStruct((N_NODES, D_OUT), jnp.float32),
    )(counts, x, w, b)


def kernel(node_features, edge_node_indices, edge_features,
           W_fn, b_fn, W_fe, b_fe, W_fa, b_fa):
    node0 = edge_node_indices[0].astype(jnp.int32)
    counts = _hist_kernel(node0)
    return _scale_matmul(counts, node_features,
                         W_fn, b_fn.reshape(1, D_OUT))


# speedup vs baseline: 1.2308x; 1.2308x over previous
import functools

import jax
import jax.numpy as jnp
from jax import lax
from jax.experimental import pallas as pl
from jax.experimental.pallas import tpu as pltpu
from jax.experimental.pallas import tpu_sc as plsc

N_NODES = 10000
N_EDGES = 320000
D_FEAT = 128
D_OUT = 128

NC = 2
NS = 16
NW = NC * NS
ROW = 100
K = 100
CPAD = 10240
FIRE = 25

_mesh = plsc.VectorSubcoreMesh(core_axis_name="c", subcore_axis_name="s",
                               num_cores=NC)


EPW = K * ROW


@functools.partial(
    pl.kernel,
    out_type=jax.ShapeDtypeStruct((NC, CPAD), jnp.float32),
    mesh=_mesh,
    scratch_types=[
        pltpu.VMEM((K, ROW), jnp.int32),
        pltpu.VMEM((112,), jnp.float32),
        pltpu.VMEM((CPAD // NS,), jnp.float32),
        pltpu.VMEM_SHARED((CPAD,), jnp.float32),
        pltpu.SemaphoreType.DMA,
    ],
)
def _hist_kernel(idx_hbm, out_hbm, idx_v, ones_v, zero_v, counts_sh, sem):
    cid = lax.axis_index("c")
    sid = lax.axis_index("s")
    wid = sid * NC + cid

    d_idx = pltpu.async_copy(idx_hbm.at[0, wid], idx_v, sem)
    for i in range(112 // 16):
        ones_v[pl.ds(i * 16, 16)] = jnp.ones((16,), jnp.float32)
    stripe = CPAD // NS
    for i in range(stripe // 16):
        zero_v[pl.ds(i * 16, 16)] = jnp.zeros((16,), jnp.float32)
    pltpu.sync_copy(zero_v, counts_sh.at[pl.ds(sid * stripe, stripe)])

    plsc.subcore_barrier()
    d_idx.wait()

    ones_row = ones_v.at[pl.ds(0, ROW)]

    def _chunk(i, carry):
        descs = []
        for b in range(FIRE):
            descs.append(pltpu.async_copy(
                ones_row, counts_sh.at[idx_v.at[i * FIRE + b]],
                sem, add=True))
        for d in descs:
            d.wait()
        return carry

    lax.fori_loop(0, K // FIRE, _chunk, 0)

    plsc.subcore_barrier()

    pltpu.sync_copy(counts_sh.at[pl.ds(sid * stripe, stripe)],
                    out_hbm.at[cid, pl.ds(sid * stripe, stripe)])


BLK = 2048


def _scale_matmul_body(c_ref, x_ref, w_ref, b_ref, o_ref):
    y = lax.dot_general(
        x_ref[...], w_ref[...],
        (((1,), (1,)), ((), ())),
        preferred_element_type=jnp.float32,
    ) + b_ref[...]
    c_col = lax.dot_general(
        c_ref[...], jnp.ones((NC, 1), jnp.float32),
        (((0,), (0,)), ((), ())),
        preferred_element_type=jnp.float32,
    )
    o_ref[...] = y * c_col


def _scale_matmul(counts, x, w, b):
    return pl.pallas_call(
        _scale_matmul_body,
        grid=(pl.cdiv(N_NODES, BLK),),
        in_specs=[
            pl.BlockSpec((NC, BLK), lambda i: (0, i)),
            pl.BlockSpec((BLK, D_FEAT), lambda i: (i, 0)),
            pl.BlockSpec((D_OUT, D_FEAT), lambda i: (0, 0)),
            pl.BlockSpec((1, D_OUT), lambda i: (0, 0)),
        ],
        out_specs=pl.BlockSpec((BLK, D_OUT), lambda i: (i, 0)),
        out_shape=jax.ShapeDtypeStruct((N_NODES, D_OUT), jnp.float32),
    )(counts, x, w, b)


def kernel(node_features, edge_node_indices, edge_features,
           W_fn, b_fn, W_fe, b_fe, W_fa, b_fa):
    idx = edge_node_indices.astype(jnp.int32).reshape(2, NW, K, ROW)
    counts = _hist_kernel(idx)
    return _scale_matmul(counts, node_features,
                         W_fn, b_fn.reshape(1, D_OUT))
